# baseline (speedup 1.0000x reference)
import functools

import jax
import jax.numpy as jnp
from jax import lax
from jax.experimental import pallas as pl
from jax.experimental.pallas import tpu as pltpu
from jax.experimental.pallas import tpu_sc as plsc

_B, _T, _C = 4, 512, 768
_G, _V = 2, 320
_TE = 64
_GV = _G * _V
_ROWS = _B * _T
_CHUNKS = _B * _G
_RPC = _ROWS // _CHUNKS
_FLAT = _RPC * _GV
_EMB = _G * _TE

_CPS = 4
_STEPS = _CHUNKS // _CPS

_NW = 32
_RPW = _ROWS // _NW
_TPC = _NW // _CHUNKS


def _tc_body(x_ref, w_ref, b_ref, prm_ref):
    s = pl.program_id(0)
    w = w_ref[...]
    bvec = b_ref[...]
    rows = lax.broadcasted_iota(jnp.int32, (_RPC, _GV), 0)
    cols = lax.broadcasted_iota(jnp.int32, (_RPC, _GV), 1)
    flat = rows * _GV + cols
    lane = lax.broadcasted_iota(jnp.int32, (1, 128), 1)
    for h in range(_CPS):
        x = x_ref[pl.ds(h * _RPC, _RPC), :]
        p = lax.dot_general(x, w, (((1,), (1,)), ((), ())),
                            preferred_element_type=jnp.float32)
        p = p + bvec
        m = jnp.max(p)
        k = jnp.min(jnp.where(p == m, flat, _FLAT))
        r = k // _GV
        c = k - r * _GV
        r_glob = (s * _CPS + h) * _RPC + r
        rowvec = jnp.where(lane == 0, r_glob, jnp.where(lane == 1, c, 0))
        prm_ref[pl.ds(s * _CPS + h, 1), :] = rowvec


def _tc_argmax(X2, W, b2):
    return pl.pallas_call(
        _tc_body,
        grid=(_STEPS,),
        in_specs=[
            pl.BlockSpec((_CPS * _RPC, _C), lambda j: (j, 0)),
            pl.BlockSpec((_GV, _C), lambda j: (0, 0)),
            pl.BlockSpec((1, _GV), lambda j: (0, 0)),
        ],
        out_specs=pl.BlockSpec((_CHUNKS, 128), lambda j: (0, 0)),
        out_shape=jax.ShapeDtypeStruct((_CHUNKS, 128), jnp.int32),
        compiler_params=pltpu.CompilerParams(
            dimension_semantics=("arbitrary",)),
    )(X2, W, b2)


@functools.partial(
    pl.kernel,
    mesh=plsc.VectorSubcoreMesh(core_axis_name="c", subcore_axis_name="s"),
    out_type=jax.ShapeDtypeStruct((_ROWS, _EMB), jnp.float32),
    scratch_types=[
        pltpu.VMEM((_RPW, _EMB), jnp.float32),
        pltpu.VMEM((_CHUNKS, 128), jnp.int32),
        pltpu.VMEM((1, _TE), jnp.float32),
    ],
)
def _sc_select(prm_hbm, cb_hbm, out_hbm, buf_v, prm_v, row_v):
    wid = lax.axis_index("s") * 2 + lax.axis_index("c")
    base = wid * _RPW
    j = wid // _TPC
    pltpu.sync_copy(prm_hbm, prm_v)
    z16 = jnp.zeros((16,), jnp.float32)

    def _zero_row(r, carry):
        for l in range(_EMB // 16):
            buf_v[r, pl.ds(l * 16, 16)] = z16
        return carry

    lax.fori_loop(0, _RPW, _zero_row, 0)

    v = prm_v[j, pl.ds(0, 16)]
    r_glob = v[0]
    c = v[1]

    @pl.when(jnp.logical_and(r_glob >= base, r_glob < base + _RPW))
    def _():
        pltpu.sync_copy(cb_hbm.at[pl.ds(c, 1)], row_v)
        local = r_glob - base
        off = (c // _V) * _TE
        for l in range(_TE // 16):
            buf_v[local, pl.ds(off + l * 16, 16)] = row_v[0, pl.ds(l * 16, 16)]

    pltpu.sync_copy(buf_v, out_hbm.at[pl.ds(base, _RPW)])


def kernel(X, W, b, codebook):
    X2 = X.reshape(_ROWS, _C)
    cb = codebook.reshape(_GV, _TE)
    b2 = b.reshape(1, _GV)
    prm = _tc_argmax(X2, W, b2)
    out = _sc_select(prm, cb)
    return out.reshape(_B, _T, _EMB)


# speedup vs baseline: 2.6506x; 2.6506x over previous
import jax
import jax.numpy as jnp
from jax import lax
from jax.experimental import pallas as pl
from jax.experimental.pallas import tpu as pltpu

_B, _T, _C = 4, 512, 768
_G, _V = 2, 320
_TE = 64
_GV = _G * _V
_ROWS = _B * _T
_CHUNKS = _B * _G
_RPC = _ROWS // _CHUNKS
_FLAT = _RPC * _GV
_EMB = _G * _TE


_CPS = 4
_STEPS = _CHUNKS // _CPS


def _vq_body(x_ref, w_ref, b_ref, cb_ref, out_ref):
    w = w_ref[...]
    bvec = b_ref[...]
    rows = lax.broadcasted_iota(jnp.int32, (_RPC, _GV), 0)
    cols = lax.broadcasted_iota(jnp.int32, (_RPC, _GV), 1)
    flat = rows * _GV + cols
    for h in range(_CPS):
        x = x_ref[pl.ds(h * _RPC, _RPC), :]
        p = lax.dot_general(x, w, (((1,), (1,)), ((), ())),
                            preferred_element_type=jnp.float32)
        p = p + bvec
        m = jnp.max(p)
        k = jnp.min(jnp.where(p == m, flat, _FLAT))
        r = k // _GV
        c = k - r * _GV
        g = c // _V
        row64 = cb_ref[pl.ds(c, 1), :]
        zero64 = jnp.zeros((1, _TE), jnp.float32)
        rowfull = jnp.concatenate(
            [jnp.where(g == 0, row64, zero64),
             jnp.where(g == 1, row64, zero64)], axis=1)
        out_ref[pl.ds(h * _RPC, _RPC), :] = jnp.zeros((_RPC, _EMB), jnp.float32)
        out_ref[pl.ds(h * _RPC + r, 1), :] = rowfull


def kernel(X, W, b, codebook):
    X2 = X.reshape(_ROWS, _C)
    cb = codebook.reshape(_GV, _TE)
    b2 = b.reshape(1, _GV)
    out = pl.pallas_call(
        _vq_body,
        grid=(_STEPS,),
        in_specs=[
            pl.BlockSpec((_CPS * _RPC, _C), lambda j: (j, 0)),
            pl.BlockSpec((_GV, _C), lambda j: (0, 0)),
            pl.BlockSpec((1, _GV), lambda j: (0, 0)),
            pl.BlockSpec((_GV, _TE), lambda j: (0, 0)),
        ],
        out_specs=pl.BlockSpec((_CPS * _RPC, _EMB), lambda j: (j, 0)),
        out_shape=jax.ShapeDtypeStruct((_ROWS, _EMB), jnp.float32),
        compiler_params=pltpu.CompilerParams(
            dimension_semantics=("parallel",)),
    )(X2, W, b2, cb)
    return out.reshape(_B, _T, _EMB)
